# baseline (speedup 1.0000x reference)
import functools

import jax
import jax.numpy as jnp
from jax import lax
from jax.experimental import pallas as pl
from jax.experimental.pallas import tpu as pltpu
from jax.experimental.pallas import tpu_sc as plsc

RES = 128
DENSITY_THRESHOLD = 0.01
N_POINTS = 2097152
N_WORDS = RES ** 3 // 32

N_WORKERS = 32
PTS_PER_WORKER = N_POINTS // N_WORKERS
CHUNK = 4096
N_CHUNKS = PTS_PER_WORKER // CHUNK


def _pack_body(g_ref, o_ref):
    i = pl.program_id(0)
    m = (g_ref[...] > DENSITY_THRESHOLD).astype(jnp.int32)
    m4 = m.reshape(8, 4, RES, RES)
    sh = lax.broadcasted_iota(jnp.int32, m4.shape, 0) + 8 * i
    part = jnp.sum(m4 << sh, axis=0)

    @pl.when(i == 0)
    def _init():
        o_ref[...] = part

    @pl.when(i > 0)
    def _acc():
        o_ref[...] |= part


_pack = pl.pallas_call(
    _pack_body,
    out_shape=jax.ShapeDtypeStruct((4, RES, RES), jnp.int32),
    grid=(4,),
    in_specs=[pl.BlockSpec((32, RES, RES), lambda i: (i, 0, 0))],
    out_specs=pl.BlockSpec((4, RES, RES), lambda i: (0, 0, 0)),
)


@functools.partial(
    pl.kernel,
    mesh=plsc.VectorSubcoreMesh(core_axis_name="c", subcore_axis_name="s"),
    out_type=jax.ShapeDtypeStruct((N_POINTS,), jnp.int32),
    compiler_params=pltpu.CompilerParams(needs_layout_passes=False),
    scratch_types=[
        pltpu.VMEM((N_WORDS,), jnp.int32),
        pltpu.VMEM((2, CHUNK), jnp.int32),
        pltpu.VMEM((2, CHUNK), jnp.int32),
        pltpu.VMEM((2, CHUNK), jnp.int32),
        pltpu.VMEM((2, CHUNK), jnp.int32),
        pltpu.SemaphoreType.DMA((2,)),
        pltpu.SemaphoreType.DMA((2,)),
    ],
)
def _sc_filter(ex_hbm, ey_hbm, ez_hbm, bits_hbm, out_hbm,
               bits_v, ex_v, ey_v, ez_v, out_v, in_sem, out_sem):
    wid = lax.axis_index("s") * 2 + lax.axis_index("c")
    base = wid * PTS_PER_WORKER

    def in_copies(ci, b):
        start = base + ci * CHUNK
        return [
            pltpu.make_async_copy(ex_hbm.at[pl.ds(start, CHUNK)], ex_v.at[b],
                                  in_sem.at[b]),
            pltpu.make_async_copy(ey_hbm.at[pl.ds(start, CHUNK)], ey_v.at[b],
                                  in_sem.at[b]),
            pltpu.make_async_copy(ez_hbm.at[pl.ds(start, CHUNK)], ez_v.at[b],
                                  in_sem.at[b]),
        ]

    def out_copy(ci, b):
        start = base + ci * CHUNK
        return pltpu.make_async_copy(out_v.at[b], out_hbm.at[pl.ds(start, CHUNK)],
                                     out_sem.at[b])

    for c in in_copies(0, 0):
        c.start()
    pltpu.sync_copy(bits_hbm, bits_v)

    def chunk_body(ci, carry):
        b = lax.rem(ci, 2)

        @pl.when(ci + 1 < N_CHUNKS)
        def _prefetch():
            for c in in_copies(ci + 1, 1 - b):
                c.start()

        for c in in_copies(ci, b):
            c.wait()

        @pl.when(ci >= 2)
        def _drain_out():
            out_copy(ci, b).wait()

        @plsc.parallel_loop(0, CHUNK, 16, unroll=8)
        def grp(o):
            ex = ex_v[b, pl.ds(o, 16)]
            ey = ey_v[b, pl.ds(o, 16)]
            ez = ez_v[b, pl.ds(o, 16)]
            f = (((ez & 127) << 7) | (ey & 127)) << 7 | (ex & 127)
            w = f & (N_WORDS - 1)
            bsh = lax.shift_right_logical(f, 16) & 31
            wv = plsc.load_gather(bits_v, [w])
            ok = 1 - lax.shift_right_logical(ex | ey | ez, 30)
            out_v[b, pl.ds(o, 16)] = lax.shift_right_logical(wv, bsh) & 1 & ok

        out_copy(ci, b).start()
        return carry

    lax.fori_loop(0, N_CHUNKS, chunk_body, None)
    out_copy(N_CHUNKS - 2, 0).wait()
    out_copy(N_CHUNKS - 1, 1).wait()


def _quant(v):
    t = (v + 1.0) * 64.0
    q = jnp.clip(t, 0.5, 127.5).astype(jnp.int32)
    return jnp.where((t >= 0.0) & (t <= 128.0), q, q | (1 << 30))


def kernel(xyz_ndc, grid):
    bits = _pack(grid).reshape(N_WORDS)
    ex = _quant(xyz_ndc[:, 0])
    ey = _quant(xyz_ndc[:, 1])
    ez = _quant(xyz_ndc[:, 2])
    out = _sc_filter(ex, ey, ez, bits)
    return out != 0


# speedup vs baseline: 1.9401x; 1.9401x over previous
import functools

import jax
import jax.numpy as jnp
from jax import lax
from jax.experimental import pallas as pl
from jax.experimental.pallas import tpu as pltpu
from jax.experimental.pallas import tpu_sc as plsc

RES = 128
DENSITY_THRESHOLD = 0.01
N_POINTS = 2097152
N_WORDS = RES ** 3 // 32

N_WORKERS = 32
PTS_PER_WORKER = N_POINTS // N_WORKERS
CHUNK = 4096
N_CHUNKS = PTS_PER_WORKER // CHUNK


def _pack_body(g_ref, o_ref):
    i = pl.program_id(0)
    m = (g_ref[...] > DENSITY_THRESHOLD).astype(jnp.int32)
    m4 = m.reshape(8, 4, RES, RES)
    sh = lax.broadcasted_iota(jnp.int32, m4.shape, 0) + 8 * i
    part = jnp.sum(m4 << sh, axis=0)

    @pl.when(i == 0)
    def _init():
        o_ref[...] = part

    @pl.when(i > 0)
    def _acc():
        o_ref[...] |= part


_pack = pl.pallas_call(
    _pack_body,
    out_shape=jax.ShapeDtypeStruct((4, RES, RES), jnp.int32),
    grid=(4,),
    in_specs=[pl.BlockSpec((32, RES, RES), lambda i: (i, 0, 0))],
    out_specs=pl.BlockSpec((4, RES, RES), lambda i: (0, 0, 0)),
)


@functools.partial(
    pl.kernel,
    mesh=plsc.VectorSubcoreMesh(core_axis_name="c", subcore_axis_name="s"),
    out_type=jax.ShapeDtypeStruct((N_POINTS,), jnp.int32),
    compiler_params=pltpu.CompilerParams(needs_layout_passes=False),
    scratch_types=[
        pltpu.VMEM((N_WORDS,), jnp.int32),
        pltpu.VMEM((2, CHUNK), jnp.float32),
        pltpu.VMEM((2, CHUNK), jnp.float32),
        pltpu.VMEM((2, CHUNK), jnp.float32),
        pltpu.VMEM((2, CHUNK), jnp.int32),
        pltpu.SemaphoreType.DMA((2,)),
        pltpu.SemaphoreType.DMA((2,)),
    ],
)
def _sc_filter(x_hbm, y_hbm, z_hbm, bits_hbm, out_hbm,
               bits_v, x_v, y_v, z_v, out_v, in_sem, out_sem):
    wid = lax.axis_index("s") * 2 + lax.axis_index("c")
    base = wid * PTS_PER_WORKER

    def in_copies(ci, b):
        start = base + ci * CHUNK
        return [
            pltpu.make_async_copy(x_hbm.at[pl.ds(start, CHUNK)], x_v.at[b],
                                  in_sem.at[b]),
            pltpu.make_async_copy(y_hbm.at[pl.ds(start, CHUNK)], y_v.at[b],
                                  in_sem.at[b]),
            pltpu.make_async_copy(z_hbm.at[pl.ds(start, CHUNK)], z_v.at[b],
                                  in_sem.at[b]),
        ]

    def out_copy(ci, b):
        start = base + ci * CHUNK
        return pltpu.make_async_copy(out_v.at[b], out_hbm.at[pl.ds(start, CHUNK)],
                                     out_sem.at[b])

    for c in in_copies(0, 0):
        c.start()
    pltpu.sync_copy(bits_hbm, bits_v)

    def chunk_body(ci, carry):
        b = lax.rem(ci, 2)

        @pl.when(ci + 1 < N_CHUNKS)
        def _prefetch():
            for c in in_copies(ci + 1, 1 - b):
                c.start()

        for c in in_copies(ci, b):
            c.wait()

        @pl.when(ci >= 2)
        def _drain_out():
            out_copy(ci, b).wait()

        @plsc.parallel_loop(0, CHUNK, 16, unroll=8)
        def grp(o):
            x = x_v[b, pl.ds(o, 16)]
            y = y_v[b, pl.ds(o, 16)]
            z = z_v[b, pl.ds(o, 16)]
            tx = (x + 1.0) * 64.0
            ty = (y + 1.0) * 64.0
            tz = (z + 1.0) * 64.0
            lo = jnp.minimum(jnp.minimum(tx, ty), tz)
            hi = jnp.maximum(jnp.maximum(tx, ty), tz)
            inb = (lo >= 0.0) & (hi <= 128.0)
            ix32 = jnp.clip(tx, 0.5, 127.5).astype(jnp.int32)
            iy32 = jnp.clip(ty, 0.5, 127.5).astype(jnp.int32)
            iz32 = jnp.clip(tz, 0.5, 127.5).astype(jnp.int32)
            f = ((iz32 << 7) | iy32) << 7 | ix32
            w = f & (N_WORDS - 1)
            bsh = lax.shift_right_logical(f, 16)
            wv = plsc.load_gather(bits_v, [w])
            bitv = lax.shift_right_logical(wv, bsh) & 1
            out_v[b, pl.ds(o, 16)] = jnp.where(inb, bitv, 0)

        out_copy(ci, b).start()
        return carry

    lax.fori_loop(0, N_CHUNKS, chunk_body, None)
    out_copy(N_CHUNKS - 2, 0).wait()
    out_copy(N_CHUNKS - 1, 1).wait()


def kernel(xyz_ndc, grid):
    bits = _pack(grid).reshape(N_WORDS)
    out = _sc_filter(xyz_ndc[:, 0], xyz_ndc[:, 1], xyz_ndc[:, 2], bits)
    return out != 0


# speedup vs baseline: 2.1744x; 1.1208x over previous
import functools

import jax
import jax.numpy as jnp
from jax import lax
from jax.experimental import pallas as pl
from jax.experimental.pallas import tpu as pltpu
from jax.experimental.pallas import tpu_sc as plsc

RES = 128
DENSITY_THRESHOLD = 0.01
N_POINTS = 2097152
N_WORDS = RES ** 3 // 32

N_WORKERS = 32
PTS_PER_WORKER = N_POINTS // N_WORKERS
CHUNK = 8192
N_CHUNKS = PTS_PER_WORKER // CHUNK

_ROWS = N_POINTS // 128


def _pack_body(g_ref, o_ref):
    i = pl.program_id(0)
    m = (g_ref[...] > DENSITY_THRESHOLD).astype(jnp.int32)
    m4 = m.reshape(8, 4, RES, RES)
    sh = lax.broadcasted_iota(jnp.int32, m4.shape, 0) + 8 * i
    part = jnp.sum(m4 << sh, axis=0)

    @pl.when(i == 0)
    def _init():
        o_ref[...] = part

    @pl.when(i > 0)
    def _acc():
        o_ref[...] |= part


_pack = pl.pallas_call(
    _pack_body,
    out_shape=jax.ShapeDtypeStruct((4, RES, RES), jnp.int32),
    grid=(4,),
    in_specs=[pl.BlockSpec((32, RES, RES), lambda i: (i, 0, 0))],
    out_specs=pl.BlockSpec((4, RES, RES), lambda i: (0, 0, 0)),
)


def _encode_body(x_ref, y_ref, z_ref, o_ref):
    tx = (x_ref[...] + 1.0) * 64.0
    ty = (y_ref[...] + 1.0) * 64.0
    tz = (z_ref[...] + 1.0) * 64.0
    lo = jnp.minimum(jnp.minimum(tx, ty), tz)
    hi = jnp.maximum(jnp.maximum(tx, ty), tz)
    inb = (lo >= 0.0) & (hi <= 128.0)
    qx = jnp.clip(tx, 0.5, 127.5).astype(jnp.int32)
    qy = jnp.clip(ty, 0.5, 127.5).astype(jnp.int32)
    qz = jnp.clip(tz, 0.5, 127.5).astype(jnp.int32)
    f = ((qz << 7) | qy) << 7 | qx
    o_ref[...] = jnp.where(inb, f, f | jnp.int32(-2 ** 31))


_ENC_BK = 2048
_encode = pl.pallas_call(
    _encode_body,
    out_shape=jax.ShapeDtypeStruct((_ROWS, 128), jnp.int32),
    grid=(_ROWS // _ENC_BK,),
    in_specs=[pl.BlockSpec((_ENC_BK, 128), lambda i: (i, 0))] * 3,
    out_specs=pl.BlockSpec((_ENC_BK, 128), lambda i: (i, 0)),
)


@functools.partial(
    pl.kernel,
    mesh=plsc.VectorSubcoreMesh(core_axis_name="c", subcore_axis_name="s"),
    out_type=jax.ShapeDtypeStruct((N_POINTS,), jnp.int32),
    compiler_params=pltpu.CompilerParams(needs_layout_passes=False),
    scratch_types=[
        pltpu.VMEM((N_WORDS,), jnp.int32),
        pltpu.VMEM((2, CHUNK), jnp.int32),
        pltpu.VMEM((2, CHUNK), jnp.int32),
        pltpu.SemaphoreType.DMA((2,)),
        pltpu.SemaphoreType.DMA((2,)),
    ],
)
def _sc_filter(enc_hbm, bits_hbm, out_hbm, bits_v, e_v, out_v, in_sem, out_sem):
    wid = lax.axis_index("s") * 2 + lax.axis_index("c")
    base = wid * PTS_PER_WORKER

    def in_copy(ci, b):
        start = base + ci * CHUNK
        return pltpu.make_async_copy(enc_hbm.at[pl.ds(start, CHUNK)], e_v.at[b],
                                     in_sem.at[b])

    def out_copy(ci, b):
        start = base + ci * CHUNK
        return pltpu.make_async_copy(out_v.at[b], out_hbm.at[pl.ds(start, CHUNK)],
                                     out_sem.at[b])

    in_copy(0, 0).start()
    pltpu.sync_copy(bits_hbm, bits_v)

    def chunk_body(ci, carry):
        b = lax.rem(ci, 2)

        @pl.when(ci + 1 < N_CHUNKS)
        def _prefetch():
            in_copy(ci + 1, 1 - b).start()

        in_copy(ci, b).wait()

        @pl.when(ci >= 2)
        def _drain_out():
            out_copy(ci, b).wait()

        @plsc.parallel_loop(0, CHUNK, 16, unroll=8)
        def grp(o):
            e = e_v[b, pl.ds(o, 16)]
            w = e & (N_WORDS - 1)
            bsh = lax.shift_right_logical(e, 16) & 31
            wv = plsc.load_gather(bits_v, [w])
            ok = jnp.bitwise_not(lax.shift_right_arithmetic(e, 31))
            out_v[b, pl.ds(o, 16)] = lax.shift_right_logical(wv, bsh) & 1 & ok

        out_copy(ci, b).start()
        return carry

    lax.fori_loop(0, N_CHUNKS, chunk_body, None)
    out_copy(N_CHUNKS - 2, 0).wait()
    out_copy(N_CHUNKS - 1, 1).wait()


def kernel(xyz_ndc, grid):
    bits = _pack(grid).reshape(N_WORDS)
    enc = _encode(xyz_ndc[:, 0].reshape(_ROWS, 128),
                  xyz_ndc[:, 1].reshape(_ROWS, 128),
                  xyz_ndc[:, 2].reshape(_ROWS, 128))
    out = _sc_filter(enc.reshape(N_POINTS), bits)
    return out != 0
